# Initial kernel scaffold; baseline (speedup 1.0000x reference)
import jax
import jax.numpy as jnp
from jax.experimental import pallas as pl


def kernel(x, edge_index, W, b):
    raise NotImplementedError("write your pallas kernel here")


# speedup vs baseline: 12.7907x; 12.7907x over previous
import functools

import jax
import jax.numpy as jnp
from jax import lax
from jax.experimental import pallas as pl
from jax.experimental.pallas import tpu as pltpu
from jax.experimental.pallas import tpu_sc as plsc

N = 10000
D = 128
DH = D // 2
NC = 2
NS = 16
EB = 128
N_ACC = 10112
RPT = N_ACC // NS
RB = 1000


def _mesh():
    return plsc.VectorSubcoreMesh(
        core_axis_name="c", subcore_axis_name="s",
        num_cores=NC, num_subcores=NS)


_SC_PARAMS = pltpu.CompilerParams(use_tc_tiling_on_sc=False)


def _zero_block(ref):
    nchunks = ref.shape[1] // 16

    def body(i, _):
        r = i // nchunks
        c = (i % nchunks) * 16
        ref[r, pl.ds(c, 16)] = jnp.zeros((16,), jnp.float32)
        return 0

    lax.fori_loop(0, 128 * nchunks, body, 0)


def _clear_shared_rows(zbuf, shared, base, rows):
    full, tail = rows // 128, rows % 128
    for c in range(full):
        pltpu.sync_copy(zbuf, shared.at[pl.ds(base + c * 128, 128)])
    if tail:
        pltpu.sync_copy(zbuf.at[pl.ds(0, tail)],
                        shared.at[pl.ds(base + full * 128, tail)])


def _make_deg_kernel(nb):
    nbh = nb // 2

    @functools.partial(
        pl.kernel,
        out_type=jax.ShapeDtypeStruct((NC, N_ACC, 16), jnp.float32),
        mesh=_mesh(),
        compiler_params=_SC_PARAMS,
        scratch_types=[
            pltpu.VMEM((nb, EB), jnp.int32),
            pltpu.VMEM((128, 16), jnp.float32),
            pltpu.VMEM_SHARED((N_ACC, 16), jnp.float32),
        ],
    )
    def deg_kernel(dst_hbm, deg_out, dst_v, buf_v, deg_sh):
        cid = lax.axis_index("c")
        sid = lax.axis_index("s")
        pltpu.sync_copy(dst_hbm.at[sid], dst_v)
        _zero_block(buf_v)
        _clear_shared_rows(buf_v, deg_sh, sid * RPT, RPT)

        def ones_body(i, _):
            buf_v[i, pl.ds(0, 16)] = jnp.ones((16,), jnp.float32)
            return 0
        lax.fori_loop(0, 128, ones_body, 0)
        plsc.subcore_barrier()

        base_b = cid * nbh

        def edge_body(b, _):
            pltpu.sync_copy(buf_v, deg_sh.at[dst_v.at[base_b + b]], add=True)
            return 0
        lax.fori_loop(0, nbh, edge_body, 0)
        plsc.subcore_barrier()
        pltpu.sync_copy(deg_sh.at[pl.ds(sid * RPT, RPT)],
                        deg_out.at[cid, pl.ds(sid * RPT, RPT)])

    return deg_kernel


def _make_spmm_kernel(nb):
    @functools.partial(
        pl.kernel,
        out_type=jax.ShapeDtypeStruct((NC, N_ACC, DH), jnp.float32),
        mesh=_mesh(),
        compiler_params=_SC_PARAMS,
        scratch_types=[
            pltpu.VMEM((nb + 1, EB), jnp.int32),
            pltpu.VMEM((nb, EB), jnp.int32),
            pltpu.VMEM((2, EB, DH), jnp.float32),
            pltpu.SemaphoreType.DMA,
            pltpu.SemaphoreType.DMA,
            pltpu.VMEM_SHARED((N_ACC, DH), jnp.float32),
        ],
    )
    def spmm_kernel(g2_hbm, src_hbm, dst_hbm, z_out,
                    src_v, dst_v, rows_v, sem0, sem1, z_sh):
        cid = lax.axis_index("c")
        sid = lax.axis_index("s")
        pltpu.sync_copy(src_hbm.at[sid], src_v)
        pltpu.sync_copy(dst_hbm.at[sid], dst_v)
        nch = EB // 16

        def xform(i, _):
            r = i // nch
            c = (i % nch) * 16
            v = src_v[r, pl.ds(c, 16)]
            src_v[r, pl.ds(c, 16)] = v * 2 + cid
            return 0
        lax.fori_loop(0, (nb + 1) * nch, xform, 0)
        _zero_block(rows_v.at[0])
        _clear_shared_rows(rows_v.at[0], z_sh, sid * RPT, RPT)
        plsc.subcore_barrier()

        sems = (sem0, sem1)
        pltpu.async_copy(g2_hbm.at[src_v.at[0]], rows_v.at[0], sem0)

        def outer(i, _):
            for b in range(2):
                j = 2 * i + b
                pltpu.async_copy(g2_hbm.at[src_v.at[j + 1]],
                                 rows_v.at[1 - b], sems[1 - b])
                pltpu.make_async_copy(g2_hbm.at[src_v.at[j]],
                                      rows_v.at[b], sems[b]).wait()
                pltpu.sync_copy(rows_v.at[b], z_sh.at[dst_v.at[j]], add=True)
            return 0

        lax.fori_loop(0, nb // 2, outer, 0)
        pltpu.make_async_copy(g2_hbm.at[src_v.at[nb]],
                              rows_v.at[0], sem0).wait()
        plsc.subcore_barrier()
        pltpu.sync_copy(z_sh.at[pl.ds(sid * RPT, RPT)],
                        z_out.at[cid, pl.ds(sid * RPT, RPT)])

    return spmm_kernel


def _deg_tot(deg_ref):
    deg = deg_ref[0] + deg_ref[1]
    return deg[:, 0:1] + 1.0


def _tc_scale0(deg_parts, x):
    def body(deg_ref, x_ref, g_ref):
        g_ref[...] = x_ref[...] * lax.rsqrt(_deg_tot(deg_ref))

    return pl.pallas_call(
        body,
        grid=(N // RB,),
        in_specs=[pl.BlockSpec((NC, RB, 16), lambda i: (0, i, 0)),
                  pl.BlockSpec((RB, D), lambda i: (i, 0))],
        out_specs=pl.BlockSpec((RB, D), lambda i: (i, 0)),
        out_shape=jax.ShapeDtypeStruct((N, D), jnp.float32),
    )(deg_parts, x)


def _tc_combine(deg_parts, z, g0):
    def body(deg_ref, z_ref, g_ref, o_ref):
        zfull = jnp.concatenate([z_ref[0], z_ref[1]], axis=1)
        o_ref[...] = (zfull + g_ref[...]) / _deg_tot(deg_ref)

    return pl.pallas_call(
        body,
        grid=(N // RB,),
        in_specs=[pl.BlockSpec((NC, RB, 16), lambda i: (0, i, 0)),
                  pl.BlockSpec((NC, RB, DH), lambda i: (0, i, 0)),
                  pl.BlockSpec((RB, D), lambda i: (i, 0))],
        out_specs=pl.BlockSpec((RB, D), lambda i: (i, 0)),
        out_shape=jax.ShapeDtypeStruct((N, D), jnp.float32),
    )(deg_parts, z, g0)


def _tc_final(deg_parts, z, g1, W, b2):
    def body(deg_ref, z_ref, g_ref, w_ref, b_ref, o_ref):
        zfull = jnp.concatenate([z_ref[0], z_ref[1]], axis=1)
        h = (zfull + g_ref[...]) * lax.rsqrt(_deg_tot(deg_ref))
        o_ref[...] = jnp.dot(h, w_ref[...],
                             preferred_element_type=jnp.float32) + b_ref[...]

    return pl.pallas_call(
        body,
        grid=(N // RB,),
        in_specs=[pl.BlockSpec((NC, RB, 16), lambda i: (0, i, 0)),
                  pl.BlockSpec((NC, RB, DH), lambda i: (0, i, 0)),
                  pl.BlockSpec((RB, D), lambda i: (i, 0)),
                  pl.BlockSpec((D, D), lambda i: (0, 0)),
                  pl.BlockSpec((1, D), lambda i: (0, 0))],
        out_specs=pl.BlockSpec((RB, D), lambda i: (i, 0)),
        out_shape=jax.ShapeDtypeStruct((N, D), jnp.float32),
    )(deg_parts, z, g1, W, b2)


def kernel(x, edge_index, W, b):
    src = edge_index[0]
    dst = edge_index[1]
    e = src.shape[0]
    nb = -(-e // (NS * EB))
    if nb % 2:
        nb += 1
    pad = NS * EB * nb - e
    src_p = jnp.concatenate([src, jnp.zeros((pad,), jnp.int32)])
    dst_p = jnp.concatenate([dst, jnp.full((pad,), N, jnp.int32)])
    src3 = src_p.reshape(NS, nb, EB)
    dst3 = dst_p.reshape(NS, nb, EB)
    srcx = jnp.concatenate([src3, jnp.zeros((NS, 1, EB), jnp.int32)], axis=1)

    deg_parts = _make_deg_kernel(nb)(dst3)
    g0 = _tc_scale0(deg_parts, x)
    spmm = _make_spmm_kernel(nb)
    z1 = spmm(g0.reshape(2 * N, DH), srcx, dst3)
    g1 = _tc_combine(deg_parts, z1, g0)
    z2 = spmm(g1.reshape(2 * N, DH), srcx, dst3)
    return _tc_final(deg_parts, z2, g1, W, jnp.reshape(b, (1, D)))
